# baseline (speedup 1.0000x reference)
import functools

import jax
import jax.numpy as jnp
from jax import lax
from jax.experimental import pallas as pl
from jax.experimental.pallas import tpu as pltpu
from jax.experimental.pallas import tpu_sc as plsc

M_EDGES = 5000
NC, NS = 2, 16
NW = NC * NS
CH = 128
ZCH = 80
_SC_PARAMS = pltpu.CompilerParams(use_tc_tiling_on_sc=False)


def _pad_seg(s):
    q = NS * ZCH
    return ((s + q) // q) * q


def _chunk_split(n_chunks):
    per, extra = divmod(n_chunks, NW)
    return per, extra


def _stage_indices(idx_hbm, idx_v, base, per, has_extra, dummy):
    pltpu.sync_copy(idx_hbm.at[pl.ds(base, per)], idx_v.at[pl.ds(0, per)])

    @pl.when(has_extra)
    def _():
        pltpu.sync_copy(idx_hbm.at[pl.ds(base + per, 1)],
                        idx_v.at[pl.ds(per, 1)])

    @pl.when(jnp.logical_not(has_extra))
    def _():
        for j in range(CH // 16):
            idx_v[per, pl.ds(j * 16, 16)] = jnp.full((16,), dummy, jnp.int32)


def _sc_degrees(v_idx2d, e_idx2d, n_pad, m_pad):
    nnz = v_idx2d.shape[0] * v_idx2d.shape[1]
    per, extra = _chunk_split(nnz // CH)
    n_my = per + 1
    mesh = plsc.VectorSubcoreMesh(core_axis_name="c", subcore_axis_name="s")

    @functools.partial(
        pl.kernel, mesh=mesh,
        out_type=[jax.ShapeDtypeStruct((NC * n_pad, 16), jnp.float32),
                  jax.ShapeDtypeStruct((NC * m_pad, 16), jnp.float32)],
        scratch_types=[
            pltpu.VMEM((n_my, CH), jnp.int32),
            pltpu.VMEM((n_my, CH), jnp.int32),
            pltpu.VMEM((CH, 16), jnp.float32),
            pltpu.VMEM_SHARED((n_pad, 16), jnp.float32),
            pltpu.VMEM_SHARED((m_pad, 16), jnp.float32),
        ],
        compiler_params=_SC_PARAMS,
    )
    def k(vi_hbm, ei_hbm, dv_hbm, de_hbm, vi_v, ei_v, ones_v, dv_sh, de_sh):
        c = lax.axis_index("c")
        s = lax.axis_index("s")
        wid = s * NC + c
        base = per * wid + jnp.minimum(wid, extra)
        has_extra = wid < extra

        _stage_indices(vi_hbm, vi_v, base, per, has_extra, n_pad - 1)
        _stage_indices(ei_hbm, ei_v, base, per, has_extra, m_pad - 1)

        @pl.loop(0, CH)
        def _(i):
            ones_v[i, pl.ds(0, 16)] = jnp.zeros((16,), jnp.float32)

        n_sub = n_pad // NS
        m_sub = m_pad // NS
        zbuf = ones_v.at[pl.ds(0, ZCH)]

        @pl.loop(0, n_sub // ZCH)
        def _(i):
            pltpu.sync_copy(zbuf, dv_sh.at[pl.ds(s * n_sub + i * ZCH, ZCH)])

        @pl.loop(0, m_sub // ZCH)
        def _(i):
            pltpu.sync_copy(zbuf, de_sh.at[pl.ds(s * m_sub + i * ZCH, ZCH)])

        @pl.loop(0, CH)
        def _(i):
            ones_v[i, pl.ds(0, 16)] = jnp.ones((16,), jnp.float32)

        plsc.subcore_barrier()

        @pl.loop(0, n_my)
        def _(i):
            pltpu.sync_copy(ones_v, dv_sh.at[vi_v.at[i]], add=True)
            pltpu.sync_copy(ones_v, de_sh.at[ei_v.at[i]], add=True)

        plsc.subcore_barrier()
        pltpu.sync_copy(dv_sh.at[pl.ds(s * n_sub, n_sub)],
                        dv_hbm.at[pl.ds(c * n_pad + s * n_sub, n_sub)])
        pltpu.sync_copy(de_sh.at[pl.ds(s * m_sub, m_sub)],
                        de_hbm.at[pl.ds(c * m_pad + s * m_sub, m_sub)])

    return k(v_idx2d, e_idx2d)


def _sc_gather_segsum(table, g_idx2d, s_idx2d, s_pad):
    t_rows, d = table.shape
    nnz = g_idx2d.shape[0] * g_idx2d.shape[1]
    per, extra = _chunk_split(nnz // CH)
    n_my = per + 1
    assert n_my % 2 == 1
    kw = 16
    last_sup = (n_my - 1) // kw * kw
    mesh = plsc.VectorSubcoreMesh(core_axis_name="c", subcore_axis_name="s")

    @functools.partial(
        pl.kernel, mesh=mesh,
        out_type=jax.ShapeDtypeStruct((NC * s_pad, d), jnp.float32),
        scratch_types=[
            pltpu.VMEM((kw, CH), jnp.int32),
            pltpu.VMEM((kw, CH), jnp.int32),
            pltpu.VMEM((CH, d), jnp.float32),
            pltpu.VMEM((CH, d), jnp.float32),
            pltpu.VMEM_SHARED((s_pad, d), jnp.float32),
            pltpu.SemaphoreType.DMA,
            pltpu.SemaphoreType.DMA,
        ],
        compiler_params=_SC_PARAMS,
    )
    def k(tab_hbm, gi_hbm, si_hbm, out_hbm, gi_v, si_v, rows0, rows1,
          acc_sh, sem0, sem1):
        c = lax.axis_index("c")
        s = lax.axis_index("s")
        wid = s * NC + c
        base = per * wid + jnp.minimum(wid, extra)
        has_extra = wid < extra

        def refill(sup):
            pltpu.sync_copy(gi_hbm.at[pl.ds(base + sup, kw)], gi_v)
            pltpu.sync_copy(si_hbm.at[pl.ds(base + sup, kw)], si_v)
            if sup == last_sup:
                @pl.when(jnp.logical_not(has_extra))
                def _():
                    for j in range(CH // 16):
                        gi_v[per % kw, pl.ds(j * 16, 16)] = (
                            jnp.zeros((16,), jnp.int32))
                        si_v[per % kw, pl.ds(j * 16, 16)] = (
                            jnp.full((16,), s_pad - 1, jnp.int32))

        @pl.loop(0, CH)
        def _(i):
            for j in range(d // 16):
                rows0[i, pl.ds(j * 16, 16)] = jnp.zeros((16,), jnp.float32)

        r_sub = s_pad // NS
        zbuf = rows0.at[pl.ds(0, ZCH)]

        @pl.loop(0, r_sub // ZCH)
        def _(i):
            pltpu.sync_copy(zbuf, acc_sh.at[pl.ds(s * r_sub + i * ZCH, ZCH)])

        plsc.subcore_barrier()

        def g_start(ch, buf, sem):
            pltpu.async_copy(tab_hbm.at[gi_v.at[ch % kw]], buf, sem)

        def g_wait(ch, buf, sem):
            pltpu.make_async_copy(tab_hbm.at[gi_v.at[ch % kw]], buf,
                                  sem).wait()

        def s_add(ch, buf):
            pltpu.sync_copy(buf, acc_sh.at[si_v.at[ch % kw]], add=True)

        refill(0)
        g_start(0, rows0, sem0)
        g_start(1, rows1, sem1)

        for sup in range(0, n_my - 1, kw):
            hi = min(sup + kw, n_my - 1)

            @pl.loop(sup, hi, step=2)
            def _(g):
                g_wait(g, rows0, sem0)
                s_add(g, rows0)

                @pl.when(g + 2 < hi)
                def _():
                    g_start(g + 2, rows0, sem0)
                g_wait(g + 1, rows1, sem1)
                s_add(g + 1, rows1)

                @pl.when(g + 3 < hi)
                def _():
                    g_start(g + 3, rows1, sem1)

            if hi % kw == 0:
                refill(hi)
            if hi < n_my:
                g_start(hi, rows0, sem0)
            if hi + 1 < n_my:
                g_start(hi + 1, rows1, sem1)

        g_wait(n_my - 1, rows0, sem0)
        s_add(n_my - 1, rows0)

        plsc.subcore_barrier()
        pltpu.sync_copy(acc_sh.at[pl.ds(s * r_sub, r_sub)],
                        out_hbm.at[pl.ds(c * s_pad + s * r_sub, r_sub)])

    return k(table, g_idx2d, s_idx2d)


def _dv_scale(dv_ref):
    col = (dv_ref[0] + dv_ref[1])[:, 0:1]
    return jnp.where(col > 0, lax.rsqrt(col), 0.0)


def _tc_mm_scale(x, w, b, dvp, bn):
    n, kdim = x.shape
    dh = w.shape[1]

    def body(x_ref, w_ref, b_ref, dv_ref, o_ref):
        h = jnp.dot(x_ref[...], w_ref[...],
                    preferred_element_type=jnp.float32) + b_ref[...]
        o_ref[...] = h * _dv_scale(dv_ref)

    return pl.pallas_call(
        body,
        grid=(n // bn,),
        in_specs=[pl.BlockSpec((bn, kdim), lambda i: (i, 0)),
                  pl.BlockSpec((kdim, dh), lambda i: (0, 0)),
                  pl.BlockSpec((1, dh), lambda i: (0, 0)),
                  pl.BlockSpec((2, bn, 16), lambda i: (0, i, 0))],
        out_specs=pl.BlockSpec((bn, dh), lambda i: (i, 0)),
        out_shape=jax.ShapeDtypeStruct((n, dh), jnp.float32),
    )(x, w, b, dvp)


def _tc_combine_descale(yp, dep, bn):
    _, m, d = yp.shape

    def body(yp_ref, de_ref, o_ref):
        col = (de_ref[0] + de_ref[1])[:, 0:1]
        inv = jnp.where(col > 0, 1.0 / col, 0.0)
        o_ref[...] = (yp_ref[0] + yp_ref[1]) * inv

    return pl.pallas_call(
        body,
        grid=(m // bn,),
        in_specs=[pl.BlockSpec((2, bn, d), lambda i: (0, i, 0)),
                  pl.BlockSpec((2, bn, 16), lambda i: (0, i, 0))],
        out_specs=pl.BlockSpec((bn, d), lambda i: (i, 0)),
        out_shape=jax.ShapeDtypeStruct((m, d), jnp.float32),
    )(yp, dep)


def _tc_layer2(xp, dvp, w2p, b2p, bn):
    _, n, d = xp.shape
    dout = w2p.shape[1]

    def body(xp_ref, dv_ref, w_ref, b_ref, o_ref):
        scale = _dv_scale(dv_ref)
        h = jnp.maximum((xp_ref[0] + xp_ref[1]) * scale, 0.0)
        y = jnp.dot(h, w_ref[...],
                    preferred_element_type=jnp.float32) + b_ref[...]
        o_ref[...] = y * scale

    return pl.pallas_call(
        body,
        grid=(n // bn,),
        in_specs=[pl.BlockSpec((2, bn, d), lambda i: (0, i, 0)),
                  pl.BlockSpec((2, bn, 16), lambda i: (0, i, 0)),
                  pl.BlockSpec((d, dout), lambda i: (0, 0)),
                  pl.BlockSpec((1, dout), lambda i: (0, 0))],
        out_specs=pl.BlockSpec((bn, dout), lambda i: (i, 0)),
        out_shape=jax.ShapeDtypeStruct((n, dout), jnp.float32),
    )(xp, dvp, w2p, b2p)


def _tc_final(xp, dvp, ncls, bn):
    _, n, d = xp.shape

    def body(xp_ref, dv_ref, o_ref):
        y = (xp_ref[0] + xp_ref[1]) * _dv_scale(dv_ref)
        o_ref[...] = y[:, :ncls]

    return pl.pallas_call(
        body,
        grid=(n // bn,),
        in_specs=[pl.BlockSpec((2, bn, d), lambda i: (0, i, 0)),
                  pl.BlockSpec((2, bn, 16), lambda i: (0, i, 0))],
        out_specs=pl.BlockSpec((bn, ncls), lambda i: (i, 0)),
        out_shape=jax.ShapeDtypeStruct((n, ncls), jnp.float32),
    )(xp, dvp)


def kernel(X, vertex_idx, hyperedge_idx, W1, b1, W2, b2):
    n, din = X.shape
    dh = W1.shape[1]
    ncls = W2.shape[1]
    m = M_EDGES
    n_pad, m_pad = _pad_seg(n), _pad_seg(m)
    d2 = ((ncls + 15) // 16) * 16

    nnz = vertex_idx.shape[0]
    v_idx = vertex_idx.astype(jnp.int32).reshape(nnz // CH, CH)
    e_idx = hyperedge_idx.astype(jnp.int32).reshape(nnz // CH, CH)
    b1r = b1.reshape(1, dh)
    w2p = jnp.pad(W2, ((0, 0), (0, d2 - ncls)))
    b2p = jnp.pad(b2, (0, d2 - ncls)).reshape(1, d2)
    bn = 1000

    dv_flat, de_flat = _sc_degrees(v_idx, e_idx, n_pad, m_pad)
    dvp = dv_flat.reshape(NC, n_pad, 16)[:, :n]
    dep = de_flat.reshape(NC, m_pad, 16)[:, :m]

    xi = _tc_mm_scale(X, W1, b1r, dvp, bn)
    yep = _sc_gather_segsum(xi, v_idx, e_idx, m_pad).reshape(NC, m_pad, dh)
    ye = _tc_combine_descale(yep[:, :m], dep, bn)
    xop = _sc_gather_segsum(ye, e_idx, v_idx, n_pad).reshape(NC, n_pad, dh)

    xi2 = _tc_layer2(xop[:, :n], dvp, w2p, b2p, bn)
    yep2 = _sc_gather_segsum(xi2, v_idx, e_idx, m_pad).reshape(NC, m_pad, d2)
    ye2 = _tc_combine_descale(yep2[:, :m], dep, bn)
    xop2 = _sc_gather_segsum(ye2, e_idx, v_idx, n_pad).reshape(NC, n_pad, d2)

    return _tc_final(xop2[:, :n], dvp, ncls, bn)


# speedup vs baseline: 1.6318x; 1.6318x over previous
import functools

import jax
import jax.numpy as jnp
from jax import lax
from jax.experimental import pallas as pl
from jax.experimental.pallas import tpu as pltpu
from jax.experimental.pallas import tpu_sc as plsc

M_EDGES = 5000
NC, NS = 2, 16
NW = NC * NS
CH = 80
ZCH = 80
_SC_PARAMS = pltpu.CompilerParams(use_tc_tiling_on_sc=False)


def _pad_seg(s):
    q = NS * ZCH
    return ((s + q) // q) * q


def _chunk_split(n_chunks):
    per, extra = divmod(n_chunks, NW)
    n_my = per + (1 if extra else 0)
    return per, extra, n_my


def _stage_indices(idx_hbm, idx_v, base, per, extra, has_extra, dummy):
    pltpu.sync_copy(idx_hbm.at[pl.ds(base, per)], idx_v.at[pl.ds(0, per)])
    if not extra:
        return

    @pl.when(has_extra)
    def _():
        pltpu.sync_copy(idx_hbm.at[pl.ds(base + per, 1)],
                        idx_v.at[pl.ds(per, 1)])

    @pl.when(jnp.logical_not(has_extra))
    def _():
        for j in range(CH // 16):
            idx_v[per, pl.ds(j * 16, 16)] = jnp.full((16,), dummy, jnp.int32)


def _sc_degrees(v_idx2d, e_idx2d, n_pad, m_pad):
    nnz = v_idx2d.shape[0] * v_idx2d.shape[1]
    per, extra, n_my = _chunk_split(nnz // CH)
    mesh = plsc.VectorSubcoreMesh(core_axis_name="c", subcore_axis_name="s")

    @functools.partial(
        pl.kernel, mesh=mesh,
        out_type=[jax.ShapeDtypeStruct((NC * n_pad, 16), jnp.float32),
                  jax.ShapeDtypeStruct((NC * m_pad, 16), jnp.float32)],
        scratch_types=[
            pltpu.VMEM((n_my, CH), jnp.int32),
            pltpu.VMEM((n_my, CH), jnp.int32),
            pltpu.VMEM((CH, 16), jnp.float32),
            pltpu.VMEM_SHARED((n_pad, 16), jnp.float32),
            pltpu.VMEM_SHARED((m_pad, 16), jnp.float32),
        ],
        compiler_params=_SC_PARAMS,
    )
    def k(vi_hbm, ei_hbm, dv_hbm, de_hbm, vi_v, ei_v, ones_v, dv_sh, de_sh):
        c = lax.axis_index("c")
        s = lax.axis_index("s")
        wid = s * NC + c
        base = per * wid + jnp.minimum(wid, extra)
        has_extra = wid < extra

        _stage_indices(vi_hbm, vi_v, base, per, extra, has_extra, n_pad - 1)
        _stage_indices(ei_hbm, ei_v, base, per, extra, has_extra, m_pad - 1)

        @pl.loop(0, CH)
        def _(i):
            ones_v[i, pl.ds(0, 16)] = jnp.zeros((16,), jnp.float32)

        n_sub = n_pad // NS
        m_sub = m_pad // NS
        zbuf = ones_v.at[pl.ds(0, ZCH)]

        @pl.loop(0, n_sub // ZCH)
        def _(i):
            pltpu.sync_copy(zbuf, dv_sh.at[pl.ds(s * n_sub + i * ZCH, ZCH)])

        @pl.loop(0, m_sub // ZCH)
        def _(i):
            pltpu.sync_copy(zbuf, de_sh.at[pl.ds(s * m_sub + i * ZCH, ZCH)])

        @pl.loop(0, CH)
        def _(i):
            ones_v[i, pl.ds(0, 16)] = jnp.ones((16,), jnp.float32)

        plsc.subcore_barrier()

        @pl.loop(0, n_my)
        def _(i):
            pltpu.sync_copy(ones_v, dv_sh.at[vi_v.at[i]], add=True)
            pltpu.sync_copy(ones_v, de_sh.at[ei_v.at[i]], add=True)

        plsc.subcore_barrier()
        pltpu.sync_copy(dv_sh.at[pl.ds(s * n_sub, n_sub)],
                        dv_hbm.at[pl.ds(c * n_pad + s * n_sub, n_sub)])
        pltpu.sync_copy(de_sh.at[pl.ds(s * m_sub, m_sub)],
                        de_hbm.at[pl.ds(c * m_pad + s * m_sub, m_sub)])

    return k(v_idx2d, e_idx2d)


def _sc_gather_segsum(table, g_idx2d, s_idx2d, s_pad):
    t_rows, d = table.shape
    nnz = g_idx2d.shape[0] * g_idx2d.shape[1]
    per, extra, n_my = _chunk_split(nnz // CH)
    assert n_my % 2 == 1
    mesh = plsc.VectorSubcoreMesh(core_axis_name="c", subcore_axis_name="s")

    @functools.partial(
        pl.kernel, mesh=mesh,
        out_type=jax.ShapeDtypeStruct((NC * s_pad, d), jnp.float32),
        scratch_types=[
            pltpu.VMEM((n_my, CH), jnp.int32),
            pltpu.VMEM((n_my, CH), jnp.int32),
            pltpu.VMEM((CH, d), jnp.float32),
            pltpu.VMEM((CH, d), jnp.float32),
            pltpu.VMEM_SHARED((s_pad, d), jnp.float32),
            pltpu.SemaphoreType.DMA,
            pltpu.SemaphoreType.DMA,
        ],
        compiler_params=_SC_PARAMS,
    )
    def k(tab_hbm, gi_hbm, si_hbm, out_hbm, gi_v, si_v, rows0, rows1,
          acc_sh, sem0, sem1):
        c = lax.axis_index("c")
        s = lax.axis_index("s")
        wid = s * NC + c
        base = per * wid + jnp.minimum(wid, extra)
        has_extra = wid < extra

        _stage_indices(gi_hbm, gi_v, base, per, extra, has_extra, 0)
        _stage_indices(si_hbm, si_v, base, per, extra, has_extra, s_pad - 1)

        @pl.loop(0, CH)
        def _(i):
            for j in range(d // 16):
                rows0[i, pl.ds(j * 16, 16)] = jnp.zeros((16,), jnp.float32)

        r_sub = s_pad // NS
        zbuf = rows0.at[pl.ds(0, ZCH)]

        @pl.loop(0, r_sub // ZCH)
        def _(i):
            pltpu.sync_copy(zbuf, acc_sh.at[pl.ds(s * r_sub + i * ZCH, ZCH)])

        plsc.subcore_barrier()

        def g_start(ch, buf, sem):
            pltpu.async_copy(tab_hbm.at[gi_v.at[ch]], buf, sem)

        def g_wait(ch, buf, sem):
            pltpu.make_async_copy(tab_hbm.at[gi_v.at[ch]], buf, sem).wait()

        def s_add(ch, buf):
            pltpu.sync_copy(buf, acc_sh.at[si_v.at[ch]], add=True)

        g_start(0, rows0, sem0)
        g_start(1, rows1, sem1)

        @pl.loop(0, n_my - 2, step=2)
        def _(g):
            g_wait(g, rows0, sem0)
            s_add(g, rows0)
            g_start(g + 2, rows0, sem0)
            g_wait(g + 1, rows1, sem1)
            s_add(g + 1, rows1)

            @pl.when(g + 3 < n_my)
            def _():
                g_start(g + 3, rows1, sem1)

        g_wait(n_my - 1, rows0, sem0)
        s_add(n_my - 1, rows0)

        plsc.subcore_barrier()
        pltpu.sync_copy(acc_sh.at[pl.ds(s * r_sub, r_sub)],
                        out_hbm.at[pl.ds(c * s_pad + s * r_sub, r_sub)])

    return k(table, g_idx2d, s_idx2d)


def _dv_scale(dv_ref):
    col = (dv_ref[0] + dv_ref[1])[:, 0:1]
    return jnp.where(col > 0, lax.rsqrt(col), 0.0)


def _tc_mm_scale(x, w, b, dvp, bn):
    n, kdim = x.shape
    dh = w.shape[1]

    def body(x_ref, w_ref, b_ref, dv_ref, o_ref):
        h = jnp.dot(x_ref[...], w_ref[...],
                    preferred_element_type=jnp.float32) + b_ref[...]
        o_ref[...] = h * _dv_scale(dv_ref)

    return pl.pallas_call(
        body,
        grid=(n // bn,),
        in_specs=[pl.BlockSpec((bn, kdim), lambda i: (i, 0)),
                  pl.BlockSpec((kdim, dh), lambda i: (0, 0)),
                  pl.BlockSpec((1, dh), lambda i: (0, 0)),
                  pl.BlockSpec((2, bn, 16), lambda i: (0, i, 0))],
        out_specs=pl.BlockSpec((bn, dh), lambda i: (i, 0)),
        out_shape=jax.ShapeDtypeStruct((n, dh), jnp.float32),
    )(x, w, b, dvp)


def _tc_combine_descale(yp, dep, bn):
    _, m, d = yp.shape

    def body(yp_ref, de_ref, o_ref):
        col = (de_ref[0] + de_ref[1])[:, 0:1]
        inv = jnp.where(col > 0, 1.0 / col, 0.0)
        o_ref[...] = (yp_ref[0] + yp_ref[1]) * inv

    return pl.pallas_call(
        body,
        grid=(m // bn,),
        in_specs=[pl.BlockSpec((2, bn, d), lambda i: (0, i, 0)),
                  pl.BlockSpec((2, bn, 16), lambda i: (0, i, 0))],
        out_specs=pl.BlockSpec((bn, d), lambda i: (i, 0)),
        out_shape=jax.ShapeDtypeStruct((m, d), jnp.float32),
    )(yp, dep)


def _tc_layer2(xp, dvp, w2p, b2p, bn):
    _, n, d = xp.shape
    dout = w2p.shape[1]

    def body(xp_ref, dv_ref, w_ref, b_ref, o_ref):
        scale = _dv_scale(dv_ref)
        h = jnp.maximum((xp_ref[0] + xp_ref[1]) * scale, 0.0)
        y = jnp.dot(h, w_ref[...],
                    preferred_element_type=jnp.float32) + b_ref[...]
        o_ref[...] = y * scale

    return pl.pallas_call(
        body,
        grid=(n // bn,),
        in_specs=[pl.BlockSpec((2, bn, d), lambda i: (0, i, 0)),
                  pl.BlockSpec((2, bn, 16), lambda i: (0, i, 0)),
                  pl.BlockSpec((d, dout), lambda i: (0, 0)),
                  pl.BlockSpec((1, dout), lambda i: (0, 0))],
        out_specs=pl.BlockSpec((bn, dout), lambda i: (i, 0)),
        out_shape=jax.ShapeDtypeStruct((n, dout), jnp.float32),
    )(xp, dvp, w2p, b2p)


def _tc_final(xp, dvp, ncls, bn):
    _, n, d = xp.shape

    def body(xp_ref, dv_ref, o_ref):
        y = (xp_ref[0] + xp_ref[1]) * _dv_scale(dv_ref)
        o_ref[...] = y[:, :ncls]

    return pl.pallas_call(
        body,
        grid=(n // bn,),
        in_specs=[pl.BlockSpec((2, bn, d), lambda i: (0, i, 0)),
                  pl.BlockSpec((2, bn, 16), lambda i: (0, i, 0))],
        out_specs=pl.BlockSpec((bn, ncls), lambda i: (i, 0)),
        out_shape=jax.ShapeDtypeStruct((n, ncls), jnp.float32),
    )(xp, dvp)


def kernel(X, vertex_idx, hyperedge_idx, W1, b1, W2, b2):
    n, din = X.shape
    dh = W1.shape[1]
    ncls = W2.shape[1]
    m = M_EDGES
    n_pad, m_pad = _pad_seg(n), _pad_seg(m)
    d2 = ((ncls + 15) // 16) * 16

    nnz = vertex_idx.shape[0]
    v_idx = vertex_idx.astype(jnp.int32).reshape(nnz // CH, CH)
    e_idx = hyperedge_idx.astype(jnp.int32).reshape(nnz // CH, CH)
    b1r = b1.reshape(1, dh)
    w2p = jnp.pad(W2, ((0, 0), (0, d2 - ncls)))
    b2p = jnp.pad(b2, (0, d2 - ncls)).reshape(1, d2)
    bn = 1000

    dv_flat, de_flat = _sc_degrees(v_idx, e_idx, n_pad, m_pad)
    dvp = dv_flat.reshape(NC, n_pad, 16)[:, :n]
    dep = de_flat.reshape(NC, m_pad, 16)[:, :m]

    xi = _tc_mm_scale(X, W1, b1r, dvp, bn)
    yep = _sc_gather_segsum(xi, v_idx, e_idx, m_pad).reshape(NC, m_pad, dh)
    ye = _tc_combine_descale(yep[:, :m], dep, bn)
    xop = _sc_gather_segsum(ye, e_idx, v_idx, n_pad).reshape(NC, n_pad, dh)

    xi2 = _tc_layer2(xop[:, :n], dvp, w2p, b2p, bn)
    yep2 = _sc_gather_segsum(xi2, v_idx, e_idx, m_pad).reshape(NC, m_pad, d2)
    ye2 = _tc_combine_descale(yep2[:, :m], dep, bn)
    xop2 = _sc_gather_segsum(ye2, e_idx, v_idx, n_pad).reshape(NC, n_pad, d2)

    return _tc_final(xop2[:, :n], dvp, ncls, bn)


# speedup vs baseline: 1.7122x; 1.0493x over previous
import functools

import jax
import jax.numpy as jnp
from jax import lax
from jax.experimental import pallas as pl
from jax.experimental.pallas import tpu as pltpu
from jax.experimental.pallas import tpu_sc as plsc

M_EDGES = 5000
NC, NS = 2, 16
NW = NC * NS
CH = 80
ZCH = 80
_SC_PARAMS = pltpu.CompilerParams(use_tc_tiling_on_sc=False)


def _pad_seg(s):
    q = NS * ZCH
    return ((s + q) // q) * q


def _chunk_split(n_chunks):
    per, extra = divmod(n_chunks, NW)
    n_my = per + (1 if extra else 0)
    return per, extra, n_my


def _stage_indices(idx_hbm, idx_v, base, per, extra, has_extra, dummy):
    pltpu.sync_copy(idx_hbm.at[pl.ds(base, per)], idx_v.at[pl.ds(0, per)])
    if not extra:
        return

    @pl.when(has_extra)
    def _():
        pltpu.sync_copy(idx_hbm.at[pl.ds(base + per, 1)],
                        idx_v.at[pl.ds(per, 1)])

    @pl.when(jnp.logical_not(has_extra))
    def _():
        for j in range(CH // 16):
            idx_v[per, pl.ds(j * 16, 16)] = jnp.full((16,), dummy, jnp.int32)


def _sc_degrees(v_idx2d, e_idx2d, n_pad, m_pad):
    nnz = v_idx2d.shape[0] * v_idx2d.shape[1]
    per, extra, n_my = _chunk_split(nnz // CH)
    mesh = plsc.VectorSubcoreMesh(core_axis_name="c", subcore_axis_name="s")

    @functools.partial(
        pl.kernel, mesh=mesh,
        out_type=[jax.ShapeDtypeStruct((NC, n_pad, 16), jnp.float32),
                  jax.ShapeDtypeStruct((NC, m_pad, 16), jnp.float32)],
        scratch_types=[
            pltpu.VMEM((n_my, CH), jnp.int32),
            pltpu.VMEM((n_my, CH), jnp.int32),
            pltpu.VMEM((CH, 16), jnp.float32),
            pltpu.VMEM_SHARED((n_pad, 16), jnp.float32),
            pltpu.VMEM_SHARED((m_pad, 16), jnp.float32),
        ],
        compiler_params=_SC_PARAMS,
    )
    def k(vi_hbm, ei_hbm, dv_hbm, de_hbm, vi_v, ei_v, ones_v, dv_sh, de_sh):
        c = lax.axis_index("c")
        s = lax.axis_index("s")
        wid = s * NC + c
        base = per * wid + jnp.minimum(wid, extra)
        has_extra = wid < extra

        _stage_indices(vi_hbm, vi_v, base, per, extra, has_extra, n_pad - 1)
        _stage_indices(ei_hbm, ei_v, base, per, extra, has_extra, m_pad - 1)

        @pl.loop(0, CH)
        def _(i):
            ones_v[i, pl.ds(0, 16)] = jnp.zeros((16,), jnp.float32)

        n_sub = n_pad // NS
        m_sub = m_pad // NS
        zbuf = ones_v.at[pl.ds(0, ZCH)]

        @pl.loop(0, n_sub // ZCH)
        def _(i):
            pltpu.sync_copy(zbuf, dv_sh.at[pl.ds(s * n_sub + i * ZCH, ZCH)])

        @pl.loop(0, m_sub // ZCH)
        def _(i):
            pltpu.sync_copy(zbuf, de_sh.at[pl.ds(s * m_sub + i * ZCH, ZCH)])

        @pl.loop(0, CH)
        def _(i):
            ones_v[i, pl.ds(0, 16)] = jnp.ones((16,), jnp.float32)

        plsc.subcore_barrier()

        @pl.loop(0, n_my)
        def _(i):
            pltpu.sync_copy(ones_v, dv_sh.at[vi_v.at[i]], add=True)
            pltpu.sync_copy(ones_v, de_sh.at[ei_v.at[i]], add=True)

        plsc.subcore_barrier()
        pltpu.sync_copy(dv_sh.at[pl.ds(s * n_sub, n_sub)],
                        dv_hbm.at[c].at[pl.ds(s * n_sub, n_sub)])
        pltpu.sync_copy(de_sh.at[pl.ds(s * m_sub, m_sub)],
                        de_hbm.at[c].at[pl.ds(s * m_sub, m_sub)])

    return k(v_idx2d, e_idx2d)


def _sc_gather_segsum(table, g_idx2d, s_idx2d, s_pad):
    t_rows, d = table.shape
    nnz = g_idx2d.shape[0] * g_idx2d.shape[1]
    per, extra, n_my = _chunk_split(nnz // CH)
    assert n_my % 2 == 1
    mesh = plsc.VectorSubcoreMesh(core_axis_name="c", subcore_axis_name="s")

    @functools.partial(
        pl.kernel, mesh=mesh,
        out_type=jax.ShapeDtypeStruct((NC, s_pad, d), jnp.float32),
        scratch_types=[
            pltpu.VMEM((n_my, CH), jnp.int32),
            pltpu.VMEM((n_my, CH), jnp.int32),
            pltpu.VMEM((CH, d), jnp.float32),
            pltpu.VMEM((CH, d), jnp.float32),
            pltpu.VMEM_SHARED((s_pad, d), jnp.float32),
            pltpu.SemaphoreType.DMA,
            pltpu.SemaphoreType.DMA,
        ],
        compiler_params=_SC_PARAMS,
    )
    def k(tab_hbm, gi_hbm, si_hbm, out_hbm, gi_v, si_v, rows0, rows1,
          acc_sh, sem0, sem1):
        c = lax.axis_index("c")
        s = lax.axis_index("s")
        wid = s * NC + c
        base = per * wid + jnp.minimum(wid, extra)
        has_extra = wid < extra

        _stage_indices(gi_hbm, gi_v, base, per, extra, has_extra, 0)
        _stage_indices(si_hbm, si_v, base, per, extra, has_extra, s_pad - 1)

        @pl.loop(0, CH)
        def _(i):
            for j in range(d // 16):
                rows0[i, pl.ds(j * 16, 16)] = jnp.zeros((16,), jnp.float32)

        r_sub = s_pad // NS
        zbuf = rows0.at[pl.ds(0, ZCH)]

        @pl.loop(0, r_sub // ZCH)
        def _(i):
            pltpu.sync_copy(zbuf, acc_sh.at[pl.ds(s * r_sub + i * ZCH, ZCH)])

        plsc.subcore_barrier()

        def g_start(ch, buf, sem):
            pltpu.async_copy(tab_hbm.at[gi_v.at[ch]], buf, sem)

        def g_wait(ch, buf, sem):
            pltpu.make_async_copy(tab_hbm.at[gi_v.at[ch]], buf, sem).wait()

        def s_add(ch, buf):
            pltpu.sync_copy(buf, acc_sh.at[si_v.at[ch]], add=True)

        g_start(0, rows0, sem0)
        g_start(1, rows1, sem1)

        @pl.loop(0, n_my - 2, step=2)
        def _(g):
            g_wait(g, rows0, sem0)
            s_add(g, rows0)
            g_start(g + 2, rows0, sem0)
            g_wait(g + 1, rows1, sem1)
            s_add(g + 1, rows1)

            @pl.when(g + 3 < n_my)
            def _():
                g_start(g + 3, rows1, sem1)

        g_wait(n_my - 1, rows0, sem0)
        s_add(n_my - 1, rows0)

        plsc.subcore_barrier()
        pltpu.sync_copy(acc_sh.at[pl.ds(s * r_sub, r_sub)],
                        out_hbm.at[c].at[pl.ds(s * r_sub, r_sub)])

    return k(table, g_idx2d, s_idx2d)


def _dv_scale(dv_ref):
    col = (dv_ref[0] + dv_ref[1])[:, 0:1]
    return jnp.where(col > 0, lax.rsqrt(col), 0.0)


def _tc_mm_scale(x, w, b, dvp, bn):
    n, kdim = x.shape
    dh = w.shape[1]

    def body(x_ref, w_ref, b_ref, dv_ref, o_ref):
        h = jnp.dot(x_ref[...], w_ref[...],
                    preferred_element_type=jnp.float32) + b_ref[...]
        o_ref[...] = h * _dv_scale(dv_ref)

    return pl.pallas_call(
        body,
        grid=(n // bn,),
        in_specs=[pl.BlockSpec((bn, kdim), lambda i: (i, 0)),
                  pl.BlockSpec((kdim, dh), lambda i: (0, 0)),
                  pl.BlockSpec((1, dh), lambda i: (0, 0)),
                  pl.BlockSpec((2, bn, 16), lambda i: (0, i, 0))],
        out_specs=pl.BlockSpec((bn, dh), lambda i: (i, 0)),
        out_shape=jax.ShapeDtypeStruct((n, dh), jnp.float32),
    )(x, w, b, dvp)


def _tc_combine_descale(yp, dep, bn, m):
    d = yp.shape[2]

    def body(yp_ref, de_ref, o_ref):
        col = (de_ref[0] + de_ref[1])[:, 0:1]
        inv = jnp.where(col > 0, 1.0 / col, 0.0)
        o_ref[...] = (yp_ref[0] + yp_ref[1]) * inv

    return pl.pallas_call(
        body,
        grid=(m // bn,),
        in_specs=[pl.BlockSpec((2, bn, d), lambda i: (0, i, 0)),
                  pl.BlockSpec((2, bn, 16), lambda i: (0, i, 0))],
        out_specs=pl.BlockSpec((bn, d), lambda i: (i, 0)),
        out_shape=jax.ShapeDtypeStruct((m, d), jnp.float32),
    )(yp, dep)


def _tc_layer2(xp, dvp, w2p, b2p, bn, n):
    d = xp.shape[2]
    dout = w2p.shape[1]

    def body(xp_ref, dv_ref, w_ref, b_ref, o_ref):
        scale = _dv_scale(dv_ref)
        h = jnp.maximum((xp_ref[0] + xp_ref[1]) * scale, 0.0)
        y = jnp.dot(h, w_ref[...],
                    preferred_element_type=jnp.float32) + b_ref[...]
        o_ref[...] = y * scale

    return pl.pallas_call(
        body,
        grid=(n // bn,),
        in_specs=[pl.BlockSpec((2, bn, d), lambda i: (0, i, 0)),
                  pl.BlockSpec((2, bn, 16), lambda i: (0, i, 0)),
                  pl.BlockSpec((d, dout), lambda i: (0, 0)),
                  pl.BlockSpec((1, dout), lambda i: (0, 0))],
        out_specs=pl.BlockSpec((bn, dout), lambda i: (i, 0)),
        out_shape=jax.ShapeDtypeStruct((n, dout), jnp.float32),
    )(xp, dvp, w2p, b2p)


def _tc_final(xp, dvp, ncls, bn, n):
    d = xp.shape[2]

    def body(xp_ref, dv_ref, o_ref):
        y = (xp_ref[0] + xp_ref[1]) * _dv_scale(dv_ref)
        o_ref[...] = y[:, :ncls]

    return pl.pallas_call(
        body,
        grid=(n // bn,),
        in_specs=[pl.BlockSpec((2, bn, d), lambda i: (0, i, 0)),
                  pl.BlockSpec((2, bn, 16), lambda i: (0, i, 0))],
        out_specs=pl.BlockSpec((bn, ncls), lambda i: (i, 0)),
        out_shape=jax.ShapeDtypeStruct((n, ncls), jnp.float32),
    )(xp, dvp)


def kernel(X, vertex_idx, hyperedge_idx, W1, b1, W2, b2):
    n, din = X.shape
    dh = W1.shape[1]
    ncls = W2.shape[1]
    m = M_EDGES
    n_pad, m_pad = _pad_seg(n), _pad_seg(m)
    d2 = ((ncls + 15) // 16) * 16

    nnz = vertex_idx.shape[0]
    v_idx = vertex_idx.astype(jnp.int32).reshape(nnz // CH, CH)
    e_idx = hyperedge_idx.astype(jnp.int32).reshape(nnz // CH, CH)
    b1r = b1.reshape(1, dh)
    w2p = jnp.pad(W2, ((0, 0), (0, d2 - ncls)))
    b2p = jnp.pad(b2, (0, d2 - ncls)).reshape(1, d2)
    bn = 1000

    dvp, dep = _sc_degrees(v_idx, e_idx, n_pad, m_pad)

    xi = _tc_mm_scale(X, W1, b1r, dvp, bn)
    yep = _sc_gather_segsum(xi, v_idx, e_idx, m_pad)
    ye = _tc_combine_descale(yep, dep, bn, m)
    xop = _sc_gather_segsum(ye, e_idx, v_idx, n_pad)

    xi2 = _tc_layer2(xop, dvp, w2p, b2p, bn, n)
    yep2 = _sc_gather_segsum(xi2, v_idx, e_idx, m_pad)
    ye2 = _tc_combine_descale(yep2, dep, bn, m)
    xop2 = _sc_gather_segsum(ye2, e_idx, v_idx, n_pad)

    return _tc_final(xop2, dvp, ncls, bn, n)


# speedup vs baseline: 1.7491x; 1.0216x over previous
import functools

import jax
import jax.numpy as jnp
from jax import lax
from jax.experimental import pallas as pl
from jax.experimental.pallas import tpu as pltpu
from jax.experimental.pallas import tpu_sc as plsc

M_EDGES = 5000
NC, NS = 2, 16
NW = NC * NS
CH = 80
ZCH = 80
_SC_PARAMS = pltpu.CompilerParams(use_tc_tiling_on_sc=False)


def _pad_seg(s):
    q = NS * ZCH
    return ((s + q) // q) * q


def _chunk_split(n_chunks):
    per, extra = divmod(n_chunks, NW)
    n_my = per + (1 if extra else 0)
    return per, extra, n_my


def _stage_indices(idx_hbm, idx_v, base, per, extra, has_extra, dummy):
    pltpu.sync_copy(idx_hbm.at[pl.ds(base, per)], idx_v.at[pl.ds(0, per)])
    if not extra:
        return

    @pl.when(has_extra)
    def _():
        pltpu.sync_copy(idx_hbm.at[pl.ds(base + per, 1)],
                        idx_v.at[pl.ds(per, 1)])

    @pl.when(jnp.logical_not(has_extra))
    def _():
        for j in range(CH // 16):
            idx_v[per, pl.ds(j * 16, 16)] = jnp.full((16,), dummy, jnp.int32)


def _sc_degrees(v_idx2d, e_idx2d, n_pad, m_pad):
    nnz = v_idx2d.shape[0] * v_idx2d.shape[1]
    per, extra, n_my = _chunk_split(nnz // CH)
    mesh = plsc.VectorSubcoreMesh(core_axis_name="c", subcore_axis_name="s")

    @functools.partial(
        pl.kernel, mesh=mesh,
        out_type=[jax.ShapeDtypeStruct((NC, n_pad, 16), jnp.float32),
                  jax.ShapeDtypeStruct((NC, m_pad, 16), jnp.float32)],
        scratch_types=[
            pltpu.VMEM((n_my, CH), jnp.int32),
            pltpu.VMEM((n_my, CH), jnp.int32),
            pltpu.VMEM((CH, 16), jnp.float32),
            pltpu.VMEM_SHARED((n_pad, 16), jnp.float32),
            pltpu.VMEM_SHARED((m_pad, 16), jnp.float32),
        ],
        compiler_params=_SC_PARAMS,
    )
    def k(vi_hbm, ei_hbm, dv_hbm, de_hbm, vi_v, ei_v, ones_v, dv_sh, de_sh):
        c = lax.axis_index("c")
        s = lax.axis_index("s")
        wid = s * NC + c
        base = per * wid + jnp.minimum(wid, extra)
        has_extra = wid < extra

        _stage_indices(vi_hbm, vi_v, base, per, extra, has_extra, n_pad - 1)
        _stage_indices(ei_hbm, ei_v, base, per, extra, has_extra, m_pad - 1)

        @pl.loop(0, CH)
        def _(i):
            ones_v[i, pl.ds(0, 16)] = jnp.zeros((16,), jnp.float32)

        n_sub = n_pad // NS
        m_sub = m_pad // NS
        zbuf = ones_v.at[pl.ds(0, ZCH)]

        @pl.loop(0, n_sub // ZCH)
        def _(i):
            pltpu.sync_copy(zbuf, dv_sh.at[pl.ds(s * n_sub + i * ZCH, ZCH)])

        @pl.loop(0, m_sub // ZCH)
        def _(i):
            pltpu.sync_copy(zbuf, de_sh.at[pl.ds(s * m_sub + i * ZCH, ZCH)])

        @pl.loop(0, CH)
        def _(i):
            ones_v[i, pl.ds(0, 16)] = jnp.ones((16,), jnp.float32)

        plsc.subcore_barrier()

        @pl.loop(0, n_my)
        def _(i):
            pltpu.sync_copy(ones_v, dv_sh.at[vi_v.at[i]], add=True)
            pltpu.sync_copy(ones_v, de_sh.at[ei_v.at[i]], add=True)

        plsc.subcore_barrier()
        pltpu.sync_copy(dv_sh.at[pl.ds(s * n_sub, n_sub)],
                        dv_hbm.at[c].at[pl.ds(s * n_sub, n_sub)])
        pltpu.sync_copy(de_sh.at[pl.ds(s * m_sub, m_sub)],
                        de_hbm.at[c].at[pl.ds(s * m_sub, m_sub)])

    return k(v_idx2d, e_idx2d)


def _sc_gather_segsum(table, g_idx2d, s_idx2d, s_pad):
    t_rows, d = table.shape
    nnz = g_idx2d.shape[0] * g_idx2d.shape[1]
    per, extra, n_my = _chunk_split(nnz // CH)
    assert n_my % 2 == 1
    mesh = plsc.VectorSubcoreMesh(core_axis_name="c", subcore_axis_name="s")

    @functools.partial(
        pl.kernel, mesh=mesh,
        out_type=jax.ShapeDtypeStruct((NC, s_pad, d), jnp.float32),
        scratch_types=[
            pltpu.VMEM((n_my, CH), jnp.int32),
            pltpu.VMEM((n_my, CH), jnp.int32),
            pltpu.VMEM((CH, d), jnp.float32),
            pltpu.VMEM((CH, d), jnp.float32),
            pltpu.VMEM_SHARED((s_pad, d), jnp.float32),
            pltpu.SemaphoreType.DMA,
            pltpu.SemaphoreType.DMA,
        ],
        compiler_params=_SC_PARAMS,
    )
    def k(tab_hbm, gi_hbm, si_hbm, out_hbm, gi_v, si_v, rows0, rows1,
          acc_sh, sem0, sem1):
        c = lax.axis_index("c")
        s = lax.axis_index("s")
        wid = s * NC + c
        base = per * wid + jnp.minimum(wid, extra)
        has_extra = wid < extra

        _stage_indices(gi_hbm, gi_v, base, per, extra, has_extra, 0)
        _stage_indices(si_hbm, si_v, base, per, extra, has_extra, s_pad - 1)

        @pl.loop(0, CH)
        def _(i):
            for j in range(d // 16):
                rows0[i, pl.ds(j * 16, 16)] = jnp.zeros((16,), jnp.float32)

        r_sub = s_pad // NS
        zbuf = rows0.at[pl.ds(0, ZCH)]

        @pl.loop(0, r_sub // ZCH)
        def _(i):
            pltpu.sync_copy(zbuf, acc_sh.at[pl.ds(s * r_sub + i * ZCH, ZCH)])

        plsc.subcore_barrier()

        def g_start(ch, buf, sem):
            pltpu.async_copy(tab_hbm.at[gi_v.at[ch]], buf, sem)

        def g_wait(ch, buf, sem):
            pltpu.make_async_copy(tab_hbm.at[gi_v.at[ch]], buf, sem).wait()

        def s_add(ch, buf):
            pltpu.sync_copy(buf, acc_sh.at[si_v.at[ch]], add=True)

        g_start(0, rows0, sem0)
        g_start(1, rows1, sem1)

        @pl.loop(0, n_my - 2, step=2)
        def _(g):
            g_wait(g, rows0, sem0)
            s_add(g, rows0)
            g_start(g + 2, rows0, sem0)
            g_wait(g + 1, rows1, sem1)
            s_add(g + 1, rows1)

            @pl.when(g + 3 < n_my)
            def _():
                g_start(g + 3, rows1, sem1)

        g_wait(n_my - 1, rows0, sem0)
        s_add(n_my - 1, rows0)

        plsc.subcore_barrier()
        pltpu.sync_copy(acc_sh.at[pl.ds(s * r_sub, r_sub)],
                        out_hbm.at[c].at[pl.ds(s * r_sub, r_sub)])

    return k(table, g_idx2d, s_idx2d)


def _dv_scale(dv_ref):
    col = (dv_ref[0] + dv_ref[1])[:, 0:1]
    return jnp.where(col > 0, lax.rsqrt(col), 0.0)


def _tc_matmul(x, w, b, bn):
    n, kdim = x.shape
    dh = w.shape[1]

    def body(x_ref, w_ref, b_ref, o_ref):
        o_ref[...] = jnp.dot(x_ref[...], w_ref[...],
                             preferred_element_type=jnp.float32) + b_ref[...]

    return pl.pallas_call(
        body,
        grid=(n // bn,),
        in_specs=[pl.BlockSpec((bn, kdim), lambda i: (i, 0)),
                  pl.BlockSpec((kdim, dh), lambda i: (0, 0)),
                  pl.BlockSpec((1, dh), lambda i: (0, 0))],
        out_specs=pl.BlockSpec((bn, dh), lambda i: (i, 0)),
        out_shape=jax.ShapeDtypeStruct((n, dh), jnp.float32),
    )(x, w, b)


def _tc_scale(h, dvp, bn):
    n, dh = h.shape

    def body(h_ref, dv_ref, o_ref):
        o_ref[...] = h_ref[...] * _dv_scale(dv_ref)

    return pl.pallas_call(
        body,
        grid=(n // bn,),
        in_specs=[pl.BlockSpec((bn, dh), lambda i: (i, 0)),
                  pl.BlockSpec((2, bn, 16), lambda i: (0, i, 0))],
        out_specs=pl.BlockSpec((bn, dh), lambda i: (i, 0)),
        out_shape=jax.ShapeDtypeStruct((n, dh), jnp.float32),
    )(h, dvp)


def _tc_combine_descale(yp, dep, bn, m):
    d = yp.shape[2]

    def body(yp_ref, de_ref, o_ref):
        col = (de_ref[0] + de_ref[1])[:, 0:1]
        inv = jnp.where(col > 0, 1.0 / col, 0.0)
        o_ref[...] = (yp_ref[0] + yp_ref[1]) * inv

    return pl.pallas_call(
        body,
        grid=(m // bn,),
        in_specs=[pl.BlockSpec((2, bn, d), lambda i: (0, i, 0)),
                  pl.BlockSpec((2, bn, 16), lambda i: (0, i, 0))],
        out_specs=pl.BlockSpec((bn, d), lambda i: (i, 0)),
        out_shape=jax.ShapeDtypeStruct((m, d), jnp.float32),
    )(yp, dep)


def _tc_layer2(xp, dvp, w2p, b2p, bn, n):
    d = xp.shape[2]
    dout = w2p.shape[1]

    def body(xp_ref, dv_ref, w_ref, b_ref, o_ref):
        scale = _dv_scale(dv_ref)
        h = jnp.maximum((xp_ref[0] + xp_ref[1]) * scale, 0.0)
        y = jnp.dot(h, w_ref[...],
                    preferred_element_type=jnp.float32) + b_ref[...]
        o_ref[...] = y * scale

    return pl.pallas_call(
        body,
        grid=(n // bn,),
        in_specs=[pl.BlockSpec((2, bn, d), lambda i: (0, i, 0)),
                  pl.BlockSpec((2, bn, 16), lambda i: (0, i, 0)),
                  pl.BlockSpec((d, dout), lambda i: (0, 0)),
                  pl.BlockSpec((1, dout), lambda i: (0, 0))],
        out_specs=pl.BlockSpec((bn, dout), lambda i: (i, 0)),
        out_shape=jax.ShapeDtypeStruct((n, dout), jnp.float32),
    )(xp, dvp, w2p, b2p)


def _tc_final(xp, dvp, ncls, bn, n):
    d = xp.shape[2]

    def body(xp_ref, dv_ref, o_ref):
        y = (xp_ref[0] + xp_ref[1]) * _dv_scale(dv_ref)
        o_ref[...] = y[:, :ncls]

    return pl.pallas_call(
        body,
        grid=(n // bn,),
        in_specs=[pl.BlockSpec((2, bn, d), lambda i: (0, i, 0)),
                  pl.BlockSpec((2, bn, 16), lambda i: (0, i, 0))],
        out_specs=pl.BlockSpec((bn, ncls), lambda i: (i, 0)),
        out_shape=jax.ShapeDtypeStruct((n, ncls), jnp.float32),
    )(xp, dvp)


def kernel(X, vertex_idx, hyperedge_idx, W1, b1, W2, b2):
    n, din = X.shape
    dh = W1.shape[1]
    ncls = W2.shape[1]
    m = M_EDGES
    n_pad, m_pad = _pad_seg(n), _pad_seg(m)
    d2 = ((ncls + 15) // 16) * 16

    nnz = vertex_idx.shape[0]
    v_idx = vertex_idx.astype(jnp.int32).reshape(nnz // CH, CH)
    e_idx = hyperedge_idx.astype(jnp.int32).reshape(nnz // CH, CH)
    b1r = b1.reshape(1, dh)
    w2p = jnp.pad(W2, ((0, 0), (0, d2 - ncls)))
    b2p = jnp.pad(b2, (0, d2 - ncls)).reshape(1, d2)
    bn = 5000

    h1 = _tc_matmul(X, W1, b1r, bn)
    dvp, dep = _sc_degrees(v_idx, e_idx, n_pad, m_pad)

    xi = _tc_scale(h1, dvp, bn)
    yep = _sc_gather_segsum(xi, v_idx, e_idx, m_pad)
    ye = _tc_combine_descale(yep, dep, bn, m)
    xop = _sc_gather_segsum(ye, e_idx, v_idx, n_pad)

    xi2 = _tc_layer2(xop, dvp, w2p, b2p, bn, n)
    yep2 = _sc_gather_segsum(xi2, v_idx, e_idx, m_pad)
    ye2 = _tc_combine_descale(yep2, dep, bn, m)
    xop2 = _sc_gather_segsum(ye2, e_idx, v_idx, n_pad)

    return _tc_final(xop2, dvp, ncls, bn, n)


# speedup vs baseline: 1.7973x; 1.0276x over previous
import functools

import jax
import jax.numpy as jnp
from jax import lax
from jax.experimental import pallas as pl
from jax.experimental.pallas import tpu as pltpu
from jax.experimental.pallas import tpu_sc as plsc

M_EDGES = 5000
NC, NS = 2, 16
NW = NC * NS
CH = 80
ZCH = 80
_SC_PARAMS = pltpu.CompilerParams(use_tc_tiling_on_sc=False)


def _pad_seg(s):
    q = NS * ZCH
    return ((s + q) // q) * q


def _chunk_split(n_chunks):
    per, extra = divmod(n_chunks, NW)
    n_my = per + (1 if extra else 0)
    return per, extra, n_my


def _stage_indices(idx_hbm, idx_v, base, per, extra, has_extra, dummy):
    pltpu.sync_copy(idx_hbm.at[pl.ds(base, per)], idx_v.at[pl.ds(0, per)])
    if not extra:
        return

    @pl.when(has_extra)
    def _():
        pltpu.sync_copy(idx_hbm.at[pl.ds(base + per, 1)],
                        idx_v.at[pl.ds(per, 1)])

    @pl.when(jnp.logical_not(has_extra))
    def _():
        for j in range(CH // 16):
            idx_v[per, pl.ds(j * 16, 16)] = jnp.full((16,), dummy, jnp.int32)


def _sc_degrees(v_idx2d, e_idx2d, n_pad, m_pad):
    nnz = v_idx2d.shape[0] * v_idx2d.shape[1]
    per, extra, n_my = _chunk_split(nnz // CH)
    mesh = plsc.VectorSubcoreMesh(core_axis_name="c", subcore_axis_name="s")

    @functools.partial(
        pl.kernel, mesh=mesh,
        out_type=[jax.ShapeDtypeStruct((NC, n_pad, 16), jnp.float32),
                  jax.ShapeDtypeStruct((NC, m_pad, 16), jnp.float32)],
        scratch_types=[
            pltpu.VMEM((n_my, CH), jnp.int32),
            pltpu.VMEM((n_my, CH), jnp.int32),
            pltpu.VMEM((CH, 16), jnp.float32),
            pltpu.VMEM_SHARED((n_pad, 16), jnp.float32),
            pltpu.VMEM_SHARED((m_pad, 16), jnp.float32),
            pltpu.SemaphoreType.DMA,
            pltpu.SemaphoreType.DMA,
        ],
        compiler_params=_SC_PARAMS,
    )
    def k(vi_hbm, ei_hbm, dv_hbm, de_hbm, vi_v, ei_v, ones_v, dv_sh, de_sh,
          sem_v, sem_e):
        c = lax.axis_index("c")
        s = lax.axis_index("s")
        wid = s * NC + c
        base = per * wid + jnp.minimum(wid, extra)
        has_extra = wid < extra

        _stage_indices(vi_hbm, vi_v, base, per, extra, has_extra, n_pad - 1)
        _stage_indices(ei_hbm, ei_v, base, per, extra, has_extra, m_pad - 1)

        @pl.loop(0, CH)
        def _(i):
            ones_v[i, pl.ds(0, 16)] = jnp.zeros((16,), jnp.float32)

        n_sub = n_pad // NS
        m_sub = m_pad // NS
        zbuf = ones_v.at[pl.ds(0, ZCH)]

        @pl.loop(0, n_sub // ZCH)
        def _(i):
            pltpu.sync_copy(zbuf, dv_sh.at[pl.ds(s * n_sub + i * ZCH, ZCH)])

        @pl.loop(0, m_sub // ZCH)
        def _(i):
            pltpu.sync_copy(zbuf, de_sh.at[pl.ds(s * m_sub + i * ZCH, ZCH)])

        @pl.loop(0, CH)
        def _(i):
            ones_v[i, pl.ds(0, 16)] = jnp.ones((16,), jnp.float32)

        plsc.subcore_barrier()

        kf = 5
        assert n_my % kf == 0

        @pl.loop(0, n_my, step=kf)
        def _(i):
            for b in range(kf):
                pltpu.async_copy(ones_v, dv_sh.at[vi_v.at[i + b]], sem_v,
                                 add=True)
                pltpu.async_copy(ones_v, de_sh.at[ei_v.at[i + b]], sem_e,
                                 add=True)
            for b in range(kf):
                pltpu.make_async_copy(ones_v, dv_sh.at[vi_v.at[i + b]],
                                      sem_v).wait()
                pltpu.make_async_copy(ones_v, de_sh.at[ei_v.at[i + b]],
                                      sem_e).wait()

        plsc.subcore_barrier()
        pltpu.sync_copy(dv_sh.at[pl.ds(s * n_sub, n_sub)],
                        dv_hbm.at[c].at[pl.ds(s * n_sub, n_sub)])
        pltpu.sync_copy(de_sh.at[pl.ds(s * m_sub, m_sub)],
                        de_hbm.at[c].at[pl.ds(s * m_sub, m_sub)])

    return k(v_idx2d, e_idx2d)


def _sc_gather_segsum(table, g_idx2d, s_idx2d, s_pad):
    t_rows, d = table.shape
    nnz = g_idx2d.shape[0] * g_idx2d.shape[1]
    per, extra, n_my = _chunk_split(nnz // CH)
    assert n_my % 2 == 1
    mesh = plsc.VectorSubcoreMesh(core_axis_name="c", subcore_axis_name="s")

    @functools.partial(
        pl.kernel, mesh=mesh,
        out_type=jax.ShapeDtypeStruct((NC, s_pad, d), jnp.float32),
        scratch_types=[
            pltpu.VMEM((n_my, CH), jnp.int32),
            pltpu.VMEM((n_my, CH), jnp.int32),
            pltpu.VMEM((CH, d), jnp.float32),
            pltpu.VMEM((CH, d), jnp.float32),
            pltpu.VMEM_SHARED((s_pad, d), jnp.float32),
            pltpu.SemaphoreType.DMA,
            pltpu.SemaphoreType.DMA,
        ],
        compiler_params=_SC_PARAMS,
    )
    def k(tab_hbm, gi_hbm, si_hbm, out_hbm, gi_v, si_v, rows0, rows1,
          acc_sh, sem0, sem1):
        c = lax.axis_index("c")
        s = lax.axis_index("s")
        wid = s * NC + c
        base = per * wid + jnp.minimum(wid, extra)
        has_extra = wid < extra

        _stage_indices(gi_hbm, gi_v, base, per, extra, has_extra, 0)
        _stage_indices(si_hbm, si_v, base, per, extra, has_extra, s_pad - 1)

        @pl.loop(0, CH)
        def _(i):
            for j in range(d // 16):
                rows0[i, pl.ds(j * 16, 16)] = jnp.zeros((16,), jnp.float32)

        r_sub = s_pad // NS
        zbuf = rows0.at[pl.ds(0, ZCH)]

        @pl.loop(0, r_sub // ZCH)
        def _(i):
            pltpu.sync_copy(zbuf, acc_sh.at[pl.ds(s * r_sub + i * ZCH, ZCH)])

        plsc.subcore_barrier()

        def g_start(ch, buf, sem):
            pltpu.async_copy(tab_hbm.at[gi_v.at[ch]], buf, sem)

        def g_wait(ch, buf, sem):
            pltpu.make_async_copy(tab_hbm.at[gi_v.at[ch]], buf, sem).wait()

        def s_add(ch, buf):
            pltpu.sync_copy(buf, acc_sh.at[si_v.at[ch]], add=True)

        g_start(0, rows0, sem0)
        g_start(1, rows1, sem1)

        @pl.loop(0, n_my - 2, step=2)
        def _(g):
            g_wait(g, rows0, sem0)
            s_add(g, rows0)
            g_start(g + 2, rows0, sem0)
            g_wait(g + 1, rows1, sem1)
            s_add(g + 1, rows1)

            @pl.when(g + 3 < n_my)
            def _():
                g_start(g + 3, rows1, sem1)

        g_wait(n_my - 1, rows0, sem0)
        s_add(n_my - 1, rows0)

        plsc.subcore_barrier()
        pltpu.sync_copy(acc_sh.at[pl.ds(s * r_sub, r_sub)],
                        out_hbm.at[c].at[pl.ds(s * r_sub, r_sub)])

    return k(table, g_idx2d, s_idx2d)


def _dv_scale(dv_ref):
    col = (dv_ref[0] + dv_ref[1])[:, 0:1]
    return jnp.where(col > 0, lax.rsqrt(col), 0.0)


def _tc_matmul(x, w, b, bn):
    n, kdim = x.shape
    dh = w.shape[1]

    def body(x_ref, w_ref, b_ref, o_ref):
        o_ref[...] = jnp.dot(x_ref[...], w_ref[...],
                             preferred_element_type=jnp.float32) + b_ref[...]

    return pl.pallas_call(
        body,
        grid=(n // bn,),
        in_specs=[pl.BlockSpec((bn, kdim), lambda i: (i, 0)),
                  pl.BlockSpec((kdim, dh), lambda i: (0, 0)),
                  pl.BlockSpec((1, dh), lambda i: (0, 0))],
        out_specs=pl.BlockSpec((bn, dh), lambda i: (i, 0)),
        out_shape=jax.ShapeDtypeStruct((n, dh), jnp.float32),
    )(x, w, b)


def _tc_scale(h, dvp, bn):
    n, dh = h.shape

    def body(h_ref, dv_ref, o_ref):
        o_ref[...] = h_ref[...] * _dv_scale(dv_ref)

    return pl.pallas_call(
        body,
        grid=(n // bn,),
        in_specs=[pl.BlockSpec((bn, dh), lambda i: (i, 0)),
                  pl.BlockSpec((2, bn, 16), lambda i: (0, i, 0))],
        out_specs=pl.BlockSpec((bn, dh), lambda i: (i, 0)),
        out_shape=jax.ShapeDtypeStruct((n, dh), jnp.float32),
    )(h, dvp)


def _tc_combine_descale(yp, dep, bn, m):
    d = yp.shape[2]

    def body(yp_ref, de_ref, o_ref):
        col = (de_ref[0] + de_ref[1])[:, 0:1]
        inv = jnp.where(col > 0, 1.0 / col, 0.0)
        o_ref[...] = (yp_ref[0] + yp_ref[1]) * inv

    return pl.pallas_call(
        body,
        grid=(m // bn,),
        in_specs=[pl.BlockSpec((2, bn, d), lambda i: (0, i, 0)),
                  pl.BlockSpec((2, bn, 16), lambda i: (0, i, 0))],
        out_specs=pl.BlockSpec((bn, d), lambda i: (i, 0)),
        out_shape=jax.ShapeDtypeStruct((m, d), jnp.float32),
    )(yp, dep)


def _tc_layer2(xp, dvp, w2p, b2p, bn, n):
    d = xp.shape[2]
    dout = w2p.shape[1]

    def body(xp_ref, dv_ref, w_ref, b_ref, o_ref):
        scale = _dv_scale(dv_ref)
        h = jnp.maximum((xp_ref[0] + xp_ref[1]) * scale, 0.0)
        y = jnp.dot(h, w_ref[...],
                    preferred_element_type=jnp.float32) + b_ref[...]
        o_ref[...] = y * scale

    return pl.pallas_call(
        body,
        grid=(n // bn,),
        in_specs=[pl.BlockSpec((2, bn, d), lambda i: (0, i, 0)),
                  pl.BlockSpec((2, bn, 16), lambda i: (0, i, 0)),
                  pl.BlockSpec((d, dout), lambda i: (0, 0)),
                  pl.BlockSpec((1, dout), lambda i: (0, 0))],
        out_specs=pl.BlockSpec((bn, dout), lambda i: (i, 0)),
        out_shape=jax.ShapeDtypeStruct((n, dout), jnp.float32),
    )(xp, dvp, w2p, b2p)


def _tc_final(xp, dvp, ncls, bn, n):
    d = xp.shape[2]

    def body(xp_ref, dv_ref, o_ref):
        y = (xp_ref[0] + xp_ref[1]) * _dv_scale(dv_ref)
        o_ref[...] = y[:, :ncls]

    return pl.pallas_call(
        body,
        grid=(n // bn,),
        in_specs=[pl.BlockSpec((2, bn, d), lambda i: (0, i, 0)),
                  pl.BlockSpec((2, bn, 16), lambda i: (0, i, 0))],
        out_specs=pl.BlockSpec((bn, ncls), lambda i: (i, 0)),
        out_shape=jax.ShapeDtypeStruct((n, ncls), jnp.float32),
    )(xp, dvp)


def kernel(X, vertex_idx, hyperedge_idx, W1, b1, W2, b2):
    n, din = X.shape
    dh = W1.shape[1]
    ncls = W2.shape[1]
    m = M_EDGES
    n_pad, m_pad = _pad_seg(n), _pad_seg(m)
    d2 = ((ncls + 15) // 16) * 16

    nnz = vertex_idx.shape[0]
    v_idx = vertex_idx.astype(jnp.int32).reshape(nnz // CH, CH)
    e_idx = hyperedge_idx.astype(jnp.int32).reshape(nnz // CH, CH)
    b1r = b1.reshape(1, dh)
    w2p = jnp.pad(W2, ((0, 0), (0, d2 - ncls)))
    b2p = jnp.pad(b2, (0, d2 - ncls)).reshape(1, d2)
    bn = 5000

    h1 = _tc_matmul(X, W1, b1r, bn)
    dvp, dep = _sc_degrees(v_idx, e_idx, n_pad, m_pad)

    xi = _tc_scale(h1, dvp, bn)
    yep = _sc_gather_segsum(xi, v_idx, e_idx, m_pad)
    ye = _tc_combine_descale(yep, dep, bn, m)
    xop = _sc_gather_segsum(ye, e_idx, v_idx, n_pad)

    xi2 = _tc_layer2(xop, dvp, w2p, b2p, bn, n)
    yep2 = _sc_gather_segsum(xi2, v_idx, e_idx, m_pad)
    ye2 = _tc_combine_descale(yep2, dep, bn, m)
    xop2 = _sc_gather_segsum(ye2, e_idx, v_idx, n_pad)

    return _tc_final(xop2, dvp, ncls, bn, n)


# speedup vs baseline: 1.8277x; 1.0169x over previous
import functools

import jax
import jax.numpy as jnp
from jax import lax
from jax.experimental import pallas as pl
from jax.experimental.pallas import tpu as pltpu
from jax.experimental.pallas import tpu_sc as plsc

M_EDGES = 5000
NC, NS = 2, 16
NW = NC * NS
CH = 80
ZCH = 80
_SC_PARAMS = pltpu.CompilerParams(use_tc_tiling_on_sc=False)


def _pad_seg(s):
    q = NS * ZCH
    return ((s + q) // q) * q


def _chunk_split(n_chunks):
    per, extra = divmod(n_chunks, NW)
    n_my = per + (1 if extra else 0)
    return per, extra, n_my


def _stage_indices(idx_hbm, idx_v, base, per, extra, has_extra, dummy):
    pltpu.sync_copy(idx_hbm.at[pl.ds(base, per)], idx_v.at[pl.ds(0, per)])
    if not extra:
        return

    @pl.when(has_extra)
    def _():
        pltpu.sync_copy(idx_hbm.at[pl.ds(base + per, 1)],
                        idx_v.at[pl.ds(per, 1)])

    @pl.when(jnp.logical_not(has_extra))
    def _():
        for j in range(CH // 16):
            idx_v[per, pl.ds(j * 16, 16)] = jnp.full((16,), dummy, jnp.int32)


def _sc_degrees(v_idx2d, e_idx2d, n_pad, m_pad):
    nnz = v_idx2d.shape[0] * v_idx2d.shape[1]
    per, extra, n_my = _chunk_split(nnz // CH)
    mesh = plsc.VectorSubcoreMesh(core_axis_name="c", subcore_axis_name="s")

    @functools.partial(
        pl.kernel, mesh=mesh,
        out_type=[jax.ShapeDtypeStruct((NC, n_pad, 16), jnp.float32),
                  jax.ShapeDtypeStruct((NC, m_pad, 16), jnp.float32)],
        scratch_types=[
            pltpu.VMEM((n_my, CH), jnp.int32),
            pltpu.VMEM((n_my, CH), jnp.int32),
            pltpu.VMEM((CH, 16), jnp.float32),
            pltpu.VMEM_SHARED((n_pad, 16), jnp.float32),
            pltpu.VMEM_SHARED((m_pad, 16), jnp.float32),
            pltpu.SemaphoreType.DMA,
            pltpu.SemaphoreType.DMA,
        ],
        compiler_params=_SC_PARAMS,
    )
    def k(vi_hbm, ei_hbm, dv_hbm, de_hbm, vi_v, ei_v, ones_v, dv_sh, de_sh,
          sem_v, sem_e):
        c = lax.axis_index("c")
        s = lax.axis_index("s")
        wid = s * NC + c
        base = per * wid + jnp.minimum(wid, extra)
        has_extra = wid < extra

        _stage_indices(vi_hbm, vi_v, base, per, extra, has_extra, n_pad - 1)
        _stage_indices(ei_hbm, ei_v, base, per, extra, has_extra, m_pad - 1)

        @pl.loop(0, CH)
        def _(i):
            ones_v[i, pl.ds(0, 16)] = jnp.zeros((16,), jnp.float32)

        n_sub = n_pad // NS
        m_sub = m_pad // NS
        zbuf = ones_v.at[pl.ds(0, ZCH)]

        @pl.loop(0, n_sub // ZCH)
        def _(i):
            pltpu.sync_copy(zbuf, dv_sh.at[pl.ds(s * n_sub + i * ZCH, ZCH)])

        @pl.loop(0, m_sub // ZCH)
        def _(i):
            pltpu.sync_copy(zbuf, de_sh.at[pl.ds(s * m_sub + i * ZCH, ZCH)])

        @pl.loop(0, CH)
        def _(i):
            ones_v[i, pl.ds(0, 16)] = jnp.ones((16,), jnp.float32)

        plsc.subcore_barrier()

        kf = 25
        assert n_my % kf == 0

        @pl.loop(0, n_my, step=kf)
        def _(i):
            for b in range(kf):
                pltpu.async_copy(ones_v, dv_sh.at[vi_v.at[i + b]], sem_v,
                                 add=True)
                pltpu.async_copy(ones_v, de_sh.at[ei_v.at[i + b]], sem_e,
                                 add=True)
            for b in range(kf):
                pltpu.make_async_copy(ones_v, dv_sh.at[vi_v.at[i + b]],
                                      sem_v).wait()
                pltpu.make_async_copy(ones_v, de_sh.at[ei_v.at[i + b]],
                                      sem_e).wait()

        plsc.subcore_barrier()
        pltpu.sync_copy(dv_sh.at[pl.ds(s * n_sub, n_sub)],
                        dv_hbm.at[c].at[pl.ds(s * n_sub, n_sub)])
        pltpu.sync_copy(de_sh.at[pl.ds(s * m_sub, m_sub)],
                        de_hbm.at[c].at[pl.ds(s * m_sub, m_sub)])

    return k(v_idx2d, e_idx2d)


def _sc_gather_segsum(table, g_idx2d, s_idx2d, s_pad):
    t_rows, d = table.shape
    nnz = g_idx2d.shape[0] * g_idx2d.shape[1]
    per, extra, n_my = _chunk_split(nnz // CH)
    assert n_my % 2 == 1
    mesh = plsc.VectorSubcoreMesh(core_axis_name="c", subcore_axis_name="s")

    @functools.partial(
        pl.kernel, mesh=mesh,
        out_type=jax.ShapeDtypeStruct((NC, s_pad, d), jnp.float32),
        scratch_types=[
            pltpu.VMEM((n_my, CH), jnp.int32),
            pltpu.VMEM((n_my, CH), jnp.int32),
            pltpu.VMEM((CH, d), jnp.float32),
            pltpu.VMEM((CH, d), jnp.float32),
            pltpu.VMEM_SHARED((s_pad, d), jnp.float32),
            pltpu.SemaphoreType.DMA,
            pltpu.SemaphoreType.DMA,
        ],
        compiler_params=_SC_PARAMS,
    )
    def k(tab_hbm, gi_hbm, si_hbm, out_hbm, gi_v, si_v, rows0, rows1,
          acc_sh, sem0, sem1):
        c = lax.axis_index("c")
        s = lax.axis_index("s")
        wid = s * NC + c
        base = per * wid + jnp.minimum(wid, extra)
        has_extra = wid < extra

        if extra == 0:
            pltpu.async_copy(gi_hbm.at[pl.ds(base, per)], gi_v, sem0)
            pltpu.async_copy(si_hbm.at[pl.ds(base, per)], si_v, sem1)
        else:
            _stage_indices(gi_hbm, gi_v, base, per, extra, has_extra, 0)
            _stage_indices(si_hbm, si_v, base, per, extra, has_extra,
                           s_pad - 1)

        @pl.loop(0, CH)
        def _(i):
            for j in range(d // 16):
                rows0[i, pl.ds(j * 16, 16)] = jnp.zeros((16,), jnp.float32)

        r_sub = s_pad // NS
        zbuf = rows0.at[pl.ds(0, ZCH)]

        @pl.loop(0, r_sub // ZCH)
        def _(i):
            pltpu.sync_copy(zbuf, acc_sh.at[pl.ds(s * r_sub + i * ZCH, ZCH)])

        if extra == 0:
            pltpu.make_async_copy(gi_hbm.at[pl.ds(base, per)], gi_v,
                                  sem0).wait()
            pltpu.make_async_copy(si_hbm.at[pl.ds(base, per)], si_v,
                                  sem1).wait()

        plsc.subcore_barrier()

        def g_start(ch, buf, sem):
            pltpu.async_copy(tab_hbm.at[gi_v.at[ch]], buf, sem)

        def g_wait(ch, buf, sem):
            pltpu.make_async_copy(tab_hbm.at[gi_v.at[ch]], buf, sem).wait()

        def s_add(ch, buf):
            pltpu.sync_copy(buf, acc_sh.at[si_v.at[ch]], add=True)

        g_start(0, rows0, sem0)
        g_start(1, rows1, sem1)

        @pl.loop(0, n_my - 2, step=2)
        def _(g):
            g_wait(g, rows0, sem0)
            s_add(g, rows0)
            g_start(g + 2, rows0, sem0)
            g_wait(g + 1, rows1, sem1)
            s_add(g + 1, rows1)

            @pl.when(g + 3 < n_my)
            def _():
                g_start(g + 3, rows1, sem1)

        g_wait(n_my - 1, rows0, sem0)
        s_add(n_my - 1, rows0)

        plsc.subcore_barrier()
        pltpu.sync_copy(acc_sh.at[pl.ds(s * r_sub, r_sub)],
                        out_hbm.at[c].at[pl.ds(s * r_sub, r_sub)])

    return k(table, g_idx2d, s_idx2d)


def _dv_scale(dv_ref):
    col = (dv_ref[0] + dv_ref[1])[:, 0:1]
    return jnp.where(col > 0, lax.rsqrt(col), 0.0)


def _tc_matmul(x, w, b, bn):
    n, kdim = x.shape
    dh = w.shape[1]

    def body(x_ref, w_ref, b_ref, o_ref):
        o_ref[...] = jnp.dot(x_ref[...], w_ref[...],
                             preferred_element_type=jnp.float32) + b_ref[...]

    return pl.pallas_call(
        body,
        grid=(n // bn,),
        in_specs=[pl.BlockSpec((bn, kdim), lambda i: (i, 0)),
                  pl.BlockSpec((kdim, dh), lambda i: (0, 0)),
                  pl.BlockSpec((1, dh), lambda i: (0, 0))],
        out_specs=pl.BlockSpec((bn, dh), lambda i: (i, 0)),
        out_shape=jax.ShapeDtypeStruct((n, dh), jnp.float32),
    )(x, w, b)


def _tc_scale(h, dvp, bn):
    n, dh = h.shape

    def body(h_ref, dv_ref, o_ref):
        o_ref[...] = h_ref[...] * _dv_scale(dv_ref)

    return pl.pallas_call(
        body,
        grid=(n // bn,),
        in_specs=[pl.BlockSpec((bn, dh), lambda i: (i, 0)),
                  pl.BlockSpec((2, bn, 16), lambda i: (0, i, 0))],
        out_specs=pl.BlockSpec((bn, dh), lambda i: (i, 0)),
        out_shape=jax.ShapeDtypeStruct((n, dh), jnp.float32),
    )(h, dvp)


def _tc_combine_descale(yp, dep, bn, m):
    d = yp.shape[2]

    def body(yp_ref, de_ref, o_ref):
        col = (de_ref[0] + de_ref[1])[:, 0:1]
        inv = jnp.where(col > 0, 1.0 / col, 0.0)
        o_ref[...] = (yp_ref[0] + yp_ref[1]) * inv

    return pl.pallas_call(
        body,
        grid=(m // bn,),
        in_specs=[pl.BlockSpec((2, bn, d), lambda i: (0, i, 0)),
                  pl.BlockSpec((2, bn, 16), lambda i: (0, i, 0))],
        out_specs=pl.BlockSpec((bn, d), lambda i: (i, 0)),
        out_shape=jax.ShapeDtypeStruct((m, d), jnp.float32),
    )(yp, dep)


def _tc_layer2(xp, dvp, w2p, b2p, bn, n):
    d = xp.shape[2]
    dout = w2p.shape[1]

    def body(xp_ref, dv_ref, w_ref, b_ref, o_ref):
        scale = _dv_scale(dv_ref)
        h = jnp.maximum((xp_ref[0] + xp_ref[1]) * scale, 0.0)
        y = jnp.dot(h, w_ref[...],
                    preferred_element_type=jnp.float32) + b_ref[...]
        o_ref[...] = y * scale

    return pl.pallas_call(
        body,
        grid=(n // bn,),
        in_specs=[pl.BlockSpec((2, bn, d), lambda i: (0, i, 0)),
                  pl.BlockSpec((2, bn, 16), lambda i: (0, i, 0)),
                  pl.BlockSpec((d, dout), lambda i: (0, 0)),
                  pl.BlockSpec((1, dout), lambda i: (0, 0))],
        out_specs=pl.BlockSpec((bn, dout), lambda i: (i, 0)),
        out_shape=jax.ShapeDtypeStruct((n, dout), jnp.float32),
    )(xp, dvp, w2p, b2p)


def _tc_final(xp, dvp, ncls, bn, n):
    d = xp.shape[2]

    def body(xp_ref, dv_ref, o_ref):
        y = (xp_ref[0] + xp_ref[1]) * _dv_scale(dv_ref)
        o_ref[...] = y[:, :ncls]

    return pl.pallas_call(
        body,
        grid=(n // bn,),
        in_specs=[pl.BlockSpec((2, bn, d), lambda i: (0, i, 0)),
                  pl.BlockSpec((2, bn, 16), lambda i: (0, i, 0))],
        out_specs=pl.BlockSpec((bn, ncls), lambda i: (i, 0)),
        out_shape=jax.ShapeDtypeStruct((n, ncls), jnp.float32),
    )(xp, dvp)


def kernel(X, vertex_idx, hyperedge_idx, W1, b1, W2, b2):
    n, din = X.shape
    dh = W1.shape[1]
    ncls = W2.shape[1]
    m = M_EDGES
    n_pad, m_pad = _pad_seg(n), _pad_seg(m)
    d2 = ((ncls + 15) // 16) * 16

    nnz = vertex_idx.shape[0]
    v_idx = vertex_idx.astype(jnp.int32).reshape(nnz // CH, CH)
    e_idx = hyperedge_idx.astype(jnp.int32).reshape(nnz // CH, CH)
    b1r = b1.reshape(1, dh)
    w2p = jnp.pad(W2, ((0, 0), (0, d2 - ncls)))
    b2p = jnp.pad(b2, (0, d2 - ncls)).reshape(1, d2)
    bn = 5000

    h1 = _tc_matmul(X, W1, b1r, bn)
    dvp, dep = _sc_degrees(v_idx, e_idx, n_pad, m_pad)

    xi = _tc_scale(h1, dvp, bn)
    yep = _sc_gather_segsum(xi, v_idx, e_idx, m_pad)
    ye = _tc_combine_descale(yep, dep, bn, m)
    xop = _sc_gather_segsum(ye, e_idx, v_idx, n_pad)

    xi2 = _tc_layer2(xop, dvp, w2p, b2p, bn, n)
    yep2 = _sc_gather_segsum(xi2, v_idx, e_idx, m_pad)
    ye2 = _tc_combine_descale(yep2, dep, bn, m)
    xop2 = _sc_gather_segsum(ye2, e_idx, v_idx, n_pad)

    return _tc_final(xop2, dvp, ncls, bn, n)
